# Initial kernel scaffold; baseline (speedup 1.0000x reference)
import functools

import jax
import jax.numpy as jnp
from jax import lax
from jax.experimental import pallas as pl
from jax.experimental.pallas import tpu as pltpu
from jax.experimental.pallas import tpu_sc as plsc

N = 10000
E = 320000
D = 128

NC = 2
NS = 16
NW = NC * NS

CHUNK = 128
NCHUNK = E // CHUNK
FULL_G = NCHUNK // NW
EXTRA = NCHUNK - FULL_G * NW
ROWS_PER_TILE = N // NS

_ZV = 2000


def _sc_body(ei_hbm, ew_hbm, x_hbm, agg_out, wsum_out,
             agg_sh, wsum_sh, rows_v, src_v, dst_v, w_v, zv, sem):
    c = lax.axis_index("c")
    s = lax.axis_index("s")
    wid = s * NC + c

    @pl.loop(0, CHUNK)
    def _(i):
        for m in range(D // 16):
            rows_v[i, pl.ds(m * 16, 16)] = jnp.zeros((16,), jnp.float32)

    @pl.loop(0, _ZV // 16)
    def _(i):
        zv[pl.ds(i * 16, 16)] = jnp.zeros((16,), jnp.float32)

    base_r = s * ROWS_PER_TILE
    off = 0
    for sz in (128, 128, 128, 128, 113):
        pltpu.sync_copy(rows_v.at[pl.ds(0, sz)],
                        agg_sh.at[pl.ds(base_r + off, sz)])
        off += sz

    @pl.when(s == 0)
    def _():
        for q in range(N // _ZV):
            pltpu.sync_copy(zv, wsum_sh.at[pl.ds(q * _ZV, _ZV)])

    plsc.subcore_barrier()

    def do_chunk(cid):
        base = cid * CHUNK
        pltpu.sync_copy(ei_hbm.at[0, pl.ds(base, CHUNK)], src_v)
        pltpu.sync_copy(ei_hbm.at[1, pl.ds(base, CHUNK)], dst_v)
        pltpu.sync_copy(ew_hbm.at[pl.ds(base, CHUNK)], w_v)
        pltpu.async_copy(x_hbm.at[src_v], rows_v, sem).wait()

        @pl.loop(0, CHUNK // 16)
        def _(k):
            w16 = w_v[pl.ds(k * 16, 16)]
            for j in range(16):
                wj = jnp.take(w16, jnp.full((16,), j, jnp.int32),
                              mode="promise_in_bounds")
                i = k * 16 + j
                for m in range(D // 16):
                    sl = pl.ds(m * 16, 16)
                    rows_v[i, sl] = rows_v[i, sl] * wj

        pltpu.sync_copy(rows_v, agg_sh.at[dst_v], add=True)
        pltpu.sync_copy(w_v, wsum_sh.at[dst_v], add=True)

    @pl.loop(0, FULL_G)
    def _(g):
        do_chunk(wid + NW * g)

    @pl.when(wid < EXTRA)
    def _():
        do_chunk(FULL_G * NW + wid)

    plsc.subcore_barrier()

    pltpu.sync_copy(agg_sh.at[pl.ds(base_r, ROWS_PER_TILE)],
                    agg_out.at[c, pl.ds(base_r, ROWS_PER_TILE)])

    @pl.when(s == 0)
    def _():
        pltpu.sync_copy(wsum_sh, wsum_out.at[c])


def _make_sc_call(interpret=False):
    mesh = plsc.VectorSubcoreMesh(core_axis_name="c", subcore_axis_name="s")
    return pl.kernel(
        _sc_body,
        out_type=(
            jax.ShapeDtypeStruct((NC, N, D), jnp.float32),
            jax.ShapeDtypeStruct((NC, N), jnp.float32),
        ),
        mesh=mesh,
        scratch_types=(
            pltpu.VMEM_SHARED((N, D), jnp.float32),
            pltpu.VMEM_SHARED((N,), jnp.float32),
            pltpu.VMEM((CHUNK, D), jnp.float32),
            pltpu.VMEM((CHUNK,), jnp.int32),
            pltpu.VMEM((CHUNK,), jnp.int32),
            pltpu.VMEM((CHUNK,), jnp.float32),
            pltpu.VMEM((_ZV,), jnp.float32),
            pltpu.SemaphoreType.DMA,
        ),
        interpret=interpret,
    )


R = 400
GRID = N // R


def _tc_body(pa_ref, pb_ref, sa_ref, sb_ref, w_ref, b_ref, o_ref):
    acc = pa_ref[0] + pb_ref[0]
    z = jnp.dot(acc, w_ref[...], preferred_element_type=jnp.float32,
                precision=lax.Precision.HIGHEST)
    svec = sa_ref[0, 0, 0] + sb_ref[0, 0, 0]
    z = z + svec[:, None] * b_ref[0][None, :]
    o_ref[...] = jnp.where(z > 0, z, jnp.expm1(z))


def _make_tc_call(interpret=False):
    return pl.pallas_call(
        _tc_body,
        grid=(GRID,),
        in_specs=[
            pl.BlockSpec((1, R, D), lambda i: (0, i, 0)),
            pl.BlockSpec((1, R, D), lambda i: (1, i, 0)),
            pl.BlockSpec((1, 1, 1, R), lambda i: (0, i, 0, 0)),
            pl.BlockSpec((1, 1, 1, R), lambda i: (1, i, 0, 0)),
            pl.BlockSpec((D, D), lambda i: (0, 0)),
            pl.BlockSpec((1, D), lambda i: (0, 0)),
        ],
        out_specs=pl.BlockSpec((R, D), lambda i: (i, 0)),
        out_shape=jax.ShapeDtypeStruct((N, D), jnp.float32),
        interpret=interpret,
    )


def _make_kernel(interpret=False):
    sc_call = _make_sc_call(interpret)
    tc_call = _make_tc_call(interpret)

    @jax.jit
    def kernel(x, edge_index, edge_weight, W, b):
        agg_parts, wsum_parts = sc_call(edge_index, edge_weight, x)
        wsum_r = wsum_parts.reshape(NC, GRID, 1, R)
        return tc_call(agg_parts, agg_parts, wsum_r, wsum_r, W,
                       b.reshape(1, D))

    return kernel


kernel = _make_kernel()


# speedup vs baseline: 4.3926x; 4.3926x over previous
import dataclasses
import functools

import jax
import jax.numpy as jnp
from jax import lax
from jax.experimental import pallas as pl
from jax.experimental.pallas import tpu as pltpu
from jax.experimental.pallas import tpu_sc as plsc

N = 10000
E = 320000
D = 128

NC = 2
NS = 16
NW = NC * NS

CHUNK = 128
NCHUNK = E // CHUNK
FULL_G = NCHUNK // NW
EXTRA = NCHUNK - FULL_G * NW
ROWS_PER_TILE = 624

_ZV = 2000


def _sc_body(ei_hbm, ew_hbm, x_hbm, agg_out, wsum_out,
             agg_sh, wsum_sh, rows_v, src_v, dst_v, w_v, zv, sem):
    c = lax.axis_index("c")
    s = lax.axis_index("s")
    wid = s * NC + c

    @pl.loop(0, CHUNK)
    def _(i):
        for m in range(D // 16):
            rows_v[i, pl.ds(m * 16, 16)] = jnp.zeros((16,), jnp.float32)

    @pl.loop(0, _ZV // 16)
    def _(i):
        zv[pl.ds(i * 16, 16)] = jnp.zeros((16,), jnp.float32)

    base_r = s * ROWS_PER_TILE
    off = 0
    for sz in (128, 128, 128, 128, 112):
        pltpu.sync_copy(rows_v.at[pl.ds(0, sz)],
                        agg_sh.at[pl.ds(base_r + off, sz)])
        off += sz

    @pl.when(s == NS - 1)
    def _():
        pltpu.sync_copy(rows_v.at[pl.ds(0, 16)],
                        agg_sh.at[pl.ds(NS * ROWS_PER_TILE, 16)])

    @pl.when(s == 0)
    def _():
        for q in range(N // _ZV):
            pltpu.sync_copy(zv, wsum_sh.at[pl.ds(q * _ZV, _ZV)])

    plsc.subcore_barrier()

    def do_chunk(cid):
        base = cid * CHUNK
        pltpu.sync_copy(ei_hbm.at[0, pl.ds(base, CHUNK)], src_v)
        pltpu.sync_copy(ei_hbm.at[1, pl.ds(base, CHUNK)], dst_v)
        pltpu.sync_copy(ew_hbm.at[pl.ds(base, CHUNK)], w_v)
        pltpu.async_copy(x_hbm.at[src_v], rows_v, sem).wait()

        @pl.loop(0, CHUNK)
        def _(i):
            wj = plsc.load_gather(w_v, [jnp.full((16,), i, jnp.int32)])
            for m in range(D // 16):
                sl = pl.ds(m * 16, 16)
                rows_v[i, sl] = rows_v[i, sl] * wj

        pltpu.sync_copy(rows_v, agg_sh.at[dst_v], add=True)
        pltpu.sync_copy(w_v, wsum_sh.at[dst_v], add=True)

    @pl.loop(0, FULL_G)
    def _(g):
        do_chunk(wid + NW * g)

    @pl.when(wid < EXTRA)
    def _():
        do_chunk(FULL_G * NW + wid)

    plsc.subcore_barrier()

    pltpu.sync_copy(agg_sh.at[pl.ds(base_r, ROWS_PER_TILE)],
                    agg_out.at[c, pl.ds(base_r, ROWS_PER_TILE)])

    @pl.when(s == NS - 1)
    def _():
        pltpu.sync_copy(agg_sh.at[pl.ds(NS * ROWS_PER_TILE, 16)],
                        agg_out.at[c, pl.ds(NS * ROWS_PER_TILE, 16)])

    @pl.when(s == 0)
    def _():
        pltpu.sync_copy(wsum_sh, wsum_out.at[c])


def _make_sc_call(interpret=False):
    mesh = plsc.VectorSubcoreMesh(core_axis_name="c", subcore_axis_name="s",
                                  num_cores=NC, num_subcores=NS)
    cp = pltpu.CompilerParams()
    if "needs_layout_passes" in pltpu.CompilerParams.__dataclass_fields__:
        cp = dataclasses.replace(cp, needs_layout_passes=False)
    return pl.kernel(
        _sc_body,
        out_type=(
            jax.ShapeDtypeStruct((NC, N, D), jnp.float32),
            jax.ShapeDtypeStruct((NC, N), jnp.float32),
        ),
        mesh=mesh,
        scratch_types=(
            pltpu.VMEM_SHARED((N, D), jnp.float32),
            pltpu.VMEM_SHARED((N,), jnp.float32),
            pltpu.VMEM((CHUNK, D), jnp.float32),
            pltpu.VMEM((CHUNK,), jnp.int32),
            pltpu.VMEM((CHUNK,), jnp.int32),
            pltpu.VMEM((CHUNK,), jnp.float32),
            pltpu.VMEM((_ZV,), jnp.float32),
            pltpu.SemaphoreType.DMA,
        ),
        compiler_params=cp,
        interpret=interpret,
    )


R = 400
GRID = N // R


def _tc_body(pa_ref, pb_ref, sa_ref, sb_ref, w_ref, b_ref, o_ref):
    acc = pa_ref[0] + pb_ref[0]
    z = jnp.dot(acc, w_ref[...], preferred_element_type=jnp.float32,
                precision=lax.Precision.HIGHEST)
    svec = sa_ref[0, 0, 0] + sb_ref[0, 0, 0]
    z = z + svec[:, None] * b_ref[0][None, :]
    o_ref[...] = jnp.where(z > 0, z, jnp.exp(z) - 1.0)


def _make_tc_call(interpret=False):
    return pl.pallas_call(
        _tc_body,
        grid=(GRID,),
        in_specs=[
            pl.BlockSpec((1, R, D), lambda i: (0, i, 0)),
            pl.BlockSpec((1, R, D), lambda i: (1, i, 0)),
            pl.BlockSpec((1, 1, 1, R), lambda i: (0, i, 0, 0)),
            pl.BlockSpec((1, 1, 1, R), lambda i: (1, i, 0, 0)),
            pl.BlockSpec((D, D), lambda i: (0, 0)),
            pl.BlockSpec((1, D), lambda i: (0, 0)),
        ],
        out_specs=pl.BlockSpec((R, D), lambda i: (i, 0)),
        out_shape=jax.ShapeDtypeStruct((N, D), jnp.float32),
        interpret=interpret,
    )


def _make_kernel(interpret=False):
    sc_call = _make_sc_call(interpret)
    tc_call = _make_tc_call(interpret)

    @jax.jit
    def kernel(x, edge_index, edge_weight, W, b):
        agg_parts, wsum_parts = sc_call(edge_index, edge_weight, x)
        wsum_r = wsum_parts.reshape(NC, GRID, 1, R)
        return tc_call(agg_parts, agg_parts, wsum_r, wsum_r, W,
                       b.reshape(1, D))

    return kernel


kernel = _make_kernel()


# speedup vs baseline: 4.9742x; 1.1324x over previous
import dataclasses
import functools

import jax
import jax.numpy as jnp
from jax import lax
from jax.experimental import pallas as pl
from jax.experimental.pallas import tpu as pltpu
from jax.experimental.pallas import tpu_sc as plsc

N = 10000
E = 320000
D = 128

NC = 2
NS = 16
NW = NC * NS

CHUNK = 128
NCHUNK = E // CHUNK
FULL_G = NCHUNK // NW
EXTRA = NCHUNK - FULL_G * NW
ROWS_PER_TILE = 624

_ZV = 2000


def _sc_body(ei_hbm, ew_hbm, x_hbm, agg_out, wsum_out,
             agg_sh, wsum_sh, rows_v, src_v, dst_v, w_v, zv, sem):
    c = lax.axis_index("c")
    s = lax.axis_index("s")
    wid = s * NC + c

    @pl.loop(0, CHUNK)
    def _(i):
        for m in range(D // 16):
            rows_v[i, pl.ds(m * 16, 16)] = jnp.zeros((16,), jnp.float32)

    @pl.loop(0, _ZV // 16)
    def _(i):
        zv[pl.ds(i * 16, 16)] = jnp.zeros((16,), jnp.float32)

    base_r = s * ROWS_PER_TILE
    off = 0
    for sz in (128, 128, 128, 128, 112):
        pltpu.sync_copy(rows_v.at[pl.ds(0, sz)],
                        agg_sh.at[pl.ds(base_r + off, sz)])
        off += sz

    @pl.when(s == NS - 1)
    def _():
        pltpu.sync_copy(rows_v.at[pl.ds(0, 16)],
                        agg_sh.at[pl.ds(NS * ROWS_PER_TILE, 16)])

    @pl.when(s == 0)
    def _():
        for q in range(N // _ZV):
            pltpu.sync_copy(zv, wsum_sh.at[pl.ds(q * _ZV, _ZV)])

    plsc.subcore_barrier()

    def do_chunk(cid):
        base = cid * CHUNK
        pltpu.sync_copy(ei_hbm.at[0, pl.ds(base, CHUNK)], src_v)
        pltpu.sync_copy(ei_hbm.at[1, pl.ds(base, CHUNK)], dst_v)
        pltpu.sync_copy(ew_hbm.at[pl.ds(base, CHUNK)], w_v)
        pltpu.async_copy(x_hbm.at[src_v], rows_v, sem).wait()

        @plsc.parallel_loop(0, CHUNK, unroll=4)
        def _(i):
            wj = plsc.load_gather(w_v, [jnp.full((16,), i, jnp.int32)])
            for m in range(D // 16):
                sl = pl.ds(m * 16, 16)
                rows_v[i, sl] = rows_v[i, sl] * wj

        pltpu.sync_copy(rows_v, agg_sh.at[dst_v], add=True)
        pltpu.sync_copy(w_v, wsum_sh.at[dst_v], add=True)

    @pl.loop(0, FULL_G)
    def _(g):
        do_chunk(wid + NW * g)

    @pl.when(wid < EXTRA)
    def _():
        do_chunk(FULL_G * NW + wid)

    plsc.subcore_barrier()

    pltpu.sync_copy(agg_sh.at[pl.ds(base_r, ROWS_PER_TILE)],
                    agg_out.at[c, pl.ds(base_r, ROWS_PER_TILE)])

    @pl.when(s == NS - 1)
    def _():
        pltpu.sync_copy(agg_sh.at[pl.ds(NS * ROWS_PER_TILE, 16)],
                        agg_out.at[c, pl.ds(NS * ROWS_PER_TILE, 16)])

    @pl.when(s == 0)
    def _():
        pltpu.sync_copy(wsum_sh, wsum_out.at[c])


def _make_sc_call(interpret=False):
    mesh = plsc.VectorSubcoreMesh(core_axis_name="c", subcore_axis_name="s",
                                  num_cores=NC, num_subcores=NS)
    cp = pltpu.CompilerParams()
    if "needs_layout_passes" in pltpu.CompilerParams.__dataclass_fields__:
        cp = dataclasses.replace(cp, needs_layout_passes=False)
    return pl.kernel(
        _sc_body,
        out_type=(
            jax.ShapeDtypeStruct((NC, N, D), jnp.float32),
            jax.ShapeDtypeStruct((NC, N), jnp.float32),
        ),
        mesh=mesh,
        scratch_types=(
            pltpu.VMEM_SHARED((N, D), jnp.float32),
            pltpu.VMEM_SHARED((N,), jnp.float32),
            pltpu.VMEM((CHUNK, D), jnp.float32),
            pltpu.VMEM((CHUNK,), jnp.int32),
            pltpu.VMEM((CHUNK,), jnp.int32),
            pltpu.VMEM((CHUNK,), jnp.float32),
            pltpu.VMEM((_ZV,), jnp.float32),
            pltpu.SemaphoreType.DMA,
        ),
        compiler_params=cp,
        interpret=interpret,
    )


R = 400
GRID = N // R


def _tc_body(pa_ref, pb_ref, sa_ref, sb_ref, w_ref, b_ref, o_ref):
    acc = pa_ref[0] + pb_ref[0]
    z = jnp.dot(acc, w_ref[...], preferred_element_type=jnp.float32,
                precision=lax.Precision.HIGHEST)
    svec = sa_ref[0, 0, 0] + sb_ref[0, 0, 0]
    z = z + svec[:, None] * b_ref[0][None, :]
    o_ref[...] = jnp.where(z > 0, z, jnp.exp(z) - 1.0)


def _make_tc_call(interpret=False):
    return pl.pallas_call(
        _tc_body,
        grid=(GRID,),
        in_specs=[
            pl.BlockSpec((1, R, D), lambda i: (0, i, 0)),
            pl.BlockSpec((1, R, D), lambda i: (1, i, 0)),
            pl.BlockSpec((1, 1, 1, R), lambda i: (0, i, 0, 0)),
            pl.BlockSpec((1, 1, 1, R), lambda i: (1, i, 0, 0)),
            pl.BlockSpec((D, D), lambda i: (0, 0)),
            pl.BlockSpec((1, D), lambda i: (0, 0)),
        ],
        out_specs=pl.BlockSpec((R, D), lambda i: (i, 0)),
        out_shape=jax.ShapeDtypeStruct((N, D), jnp.float32),
        interpret=interpret,
    )


def _make_kernel(interpret=False):
    sc_call = _make_sc_call(interpret)
    tc_call = _make_tc_call(interpret)

    @jax.jit
    def kernel(x, edge_index, edge_weight, W, b):
        agg_parts, wsum_parts = sc_call(edge_index, edge_weight, x)
        wsum_r = wsum_parts.reshape(NC, GRID, 1, R)
        return tc_call(agg_parts, agg_parts, wsum_r, wsum_r, W,
                       b.reshape(1, D))

    return kernel


kernel = _make_kernel()


# speedup vs baseline: 9.6902x; 1.9481x over previous
import dataclasses
import functools

import jax
import jax.numpy as jnp
from jax import lax
from jax.experimental import pallas as pl
from jax.experimental.pallas import tpu as pltpu
from jax.experimental.pallas import tpu_sc as plsc

N = 10000
E = 320000
D = 128

NC = 2
NS = 16
NW = NC * NS

CHUNK = 128
NCHUNK = E // CHUNK
FULL_G = NCHUNK // NW
EXTRA = NCHUNK - FULL_G * NW
ROWS_PER_TILE = 624

_ZV = 2000


NBUF = 3


def _sc_body(ei_hbm, ew_hbm, x_hbm, agg_out, wsum_out,
             agg_sh, wsum_sh,
             rows0, rows1, rows2, idx0, idx1, idx2, w0, w1, w2,
             sg0, sg1, sg2, sr0, sr1, sr2, sw0, sw1, sw2):
    rows = (rows0, rows1, rows2)
    idx = (idx0, idx1, idx2)
    wv = (w0, w1, w2)
    sg = (sg0, sg1, sg2)
    sr = (sr0, sr1, sr2)
    sw = (sw0, sw1, sw2)
    rows_v = rows0

    c = lax.axis_index("c")
    s = lax.axis_index("s")
    wid = s * NC + c

    @pl.loop(0, CHUNK)
    def _(i):
        for m in range(D // 16):
            rows_v[i, pl.ds(m * 16, 16)] = jnp.zeros((16,), jnp.float32)

    for m in range(CHUNK // 16):
        w0[pl.ds(m * 16, 16)] = jnp.zeros((16,), jnp.float32)

    base_r = s * ROWS_PER_TILE
    off = 0
    for sz in (128, 128, 128, 128, 112):
        pltpu.sync_copy(rows_v.at[pl.ds(0, sz)],
                        agg_sh.at[pl.ds(base_r + off, sz)])
        off += sz

    @pl.when(s == NS - 1)
    def _():
        pltpu.sync_copy(rows_v.at[pl.ds(0, 16)],
                        agg_sh.at[pl.ds(NS * ROWS_PER_TILE, 16)])

    woff = 0
    for wsz in (128, 128, 128, 128, 112):
        pltpu.sync_copy(w0.at[pl.ds(0, wsz)],
                        wsum_sh.at[pl.ds(base_r + woff, wsz)])
        woff += wsz

    @pl.when(s == NS - 1)
    def _():
        pltpu.sync_copy(w0.at[pl.ds(0, 16)],
                        wsum_sh.at[pl.ds(NS * ROWS_PER_TILE, 16)])

    plsc.subcore_barrier()

    def idx_load(cid, b):
        base = cid * CHUNK
        pltpu.sync_copy(ei_hbm.at[:, pl.ds(base, CHUNK)], idx[b])
        pltpu.sync_copy(ew_hbm.at[pl.ds(base, CHUNK)], wv[b])

    def gather_start(b):
        pltpu.async_copy(x_hbm.at[idx[b].at[0]], rows[b], sg[b])

    def gather_wait(b):
        pltpu.make_async_copy(x_hbm.at[idx[b].at[0]], rows[b], sg[b]).wait()

    def scale(b):
        rb = rows[b]
        wb = wv[b]

        @plsc.parallel_loop(0, CHUNK, unroll=4)
        def _(i):
            wj = plsc.load_gather(wb, [jnp.full((16,), i, jnp.int32)])
            for m in range(D // 16):
                sl = pl.ds(m * 16, 16)
                rb[i, sl] = rb[i, sl] * wj

    def scatter_start(b):
        pltpu.async_copy(rows[b], agg_sh.at[idx[b].at[1]], sr[b], add=True)
        pltpu.async_copy(wv[b], wsum_sh.at[idx[b].at[1]], sw[b], add=True)

    def scatter_wait(b):
        pltpu.make_async_copy(rows[b], agg_sh.at[idx[b].at[1]], sr[b]).wait()
        pltpu.make_async_copy(wv[b], wsum_sh.at[idx[b].at[1]], sw[b]).wait()

    idx_load(wid, 0)
    gather_start(0)

    @pl.loop(0, FULL_G // NBUF)
    def _(G):
        for u in range(NBUF):
            cc = NBUF * G + u
            b = u
            b1 = (u + 1) % NBUF

            @pl.when(cc < FULL_G - 1)
            def _():
                @pl.when(cc >= 2)
                def _():
                    scatter_wait(b1)
                idx_load(wid + NW * (cc + 1), b1)
                gather_start(b1)

            gather_wait(b)
            scale(b)
            scatter_start(b)

    @pl.when(wid < EXTRA)
    def _():
        scatter_wait(0)
        idx_load(FULL_G * NW + wid, 0)
        gather_start(0)
        gather_wait(0)
        scale(0)
        scatter_start(0)

    scatter_wait(1)
    scatter_wait(2)
    scatter_wait(0)

    plsc.subcore_barrier()

    pltpu.sync_copy(agg_sh.at[pl.ds(base_r, ROWS_PER_TILE)],
                    agg_out.at[c, pl.ds(base_r, ROWS_PER_TILE)])

    @pl.when(s == NS - 1)
    def _():
        pltpu.sync_copy(agg_sh.at[pl.ds(NS * ROWS_PER_TILE, 16)],
                        agg_out.at[c, pl.ds(NS * ROWS_PER_TILE, 16)])

    @pl.when(s == 0)
    def _():
        pltpu.sync_copy(wsum_sh, wsum_out.at[c])


def _make_sc_call(interpret=False):
    mesh = plsc.VectorSubcoreMesh(core_axis_name="c", subcore_axis_name="s",
                                  num_cores=NC, num_subcores=NS)
    cp = pltpu.CompilerParams()
    if "needs_layout_passes" in pltpu.CompilerParams.__dataclass_fields__:
        cp = dataclasses.replace(cp, needs_layout_passes=False)
    return pl.kernel(
        _sc_body,
        out_type=(
            jax.ShapeDtypeStruct((NC, N, D), jnp.float32),
            jax.ShapeDtypeStruct((NC, N), jnp.float32),
        ),
        mesh=mesh,
        scratch_types=(
            pltpu.VMEM_SHARED((N, D), jnp.float32),
            pltpu.VMEM_SHARED((N,), jnp.float32),
            pltpu.VMEM((CHUNK, D), jnp.float32),
            pltpu.VMEM((CHUNK, D), jnp.float32),
            pltpu.VMEM((CHUNK, D), jnp.float32),
            pltpu.VMEM((2, CHUNK), jnp.int32),
            pltpu.VMEM((2, CHUNK), jnp.int32),
            pltpu.VMEM((2, CHUNK), jnp.int32),
            pltpu.VMEM((CHUNK,), jnp.float32),
            pltpu.VMEM((CHUNK,), jnp.float32),
            pltpu.VMEM((CHUNK,), jnp.float32),
            pltpu.SemaphoreType.DMA,
            pltpu.SemaphoreType.DMA,
            pltpu.SemaphoreType.DMA,
            pltpu.SemaphoreType.DMA,
            pltpu.SemaphoreType.DMA,
            pltpu.SemaphoreType.DMA,
            pltpu.SemaphoreType.DMA,
            pltpu.SemaphoreType.DMA,
            pltpu.SemaphoreType.DMA,
        ),
        compiler_params=cp,
        interpret=interpret,
    )


R = 400
GRID = N // R


def _tc_body(pa_ref, pb_ref, sa_ref, sb_ref, w_ref, b_ref, o_ref):
    acc = pa_ref[0] + pb_ref[0]
    z = jnp.dot(acc, w_ref[...], preferred_element_type=jnp.float32,
                precision=lax.Precision.HIGHEST)
    svec = sa_ref[0, 0, 0] + sb_ref[0, 0, 0]
    z = z + svec[:, None] * b_ref[0][None, :]
    o_ref[...] = jnp.where(z > 0, z, jnp.exp(z) - 1.0)


def _make_tc_call(interpret=False):
    return pl.pallas_call(
        _tc_body,
        grid=(GRID,),
        in_specs=[
            pl.BlockSpec((1, R, D), lambda i: (0, i, 0)),
            pl.BlockSpec((1, R, D), lambda i: (1, i, 0)),
            pl.BlockSpec((1, 1, 1, R), lambda i: (0, i, 0, 0)),
            pl.BlockSpec((1, 1, 1, R), lambda i: (1, i, 0, 0)),
            pl.BlockSpec((D, D), lambda i: (0, 0)),
            pl.BlockSpec((1, D), lambda i: (0, 0)),
        ],
        out_specs=pl.BlockSpec((R, D), lambda i: (i, 0)),
        out_shape=jax.ShapeDtypeStruct((N, D), jnp.float32),
        interpret=interpret,
    )


def _make_kernel(interpret=False):
    sc_call = _make_sc_call(interpret)
    tc_call = _make_tc_call(interpret)

    @jax.jit
    def kernel(x, edge_index, edge_weight, W, b):
        agg_parts, wsum_parts = sc_call(edge_index, edge_weight, x)
        wsum_r = wsum_parts.reshape(NC, GRID, 1, R)
        return tc_call(agg_parts, agg_parts, wsum_r, wsum_r, W,
                       b.reshape(1, D))

    return kernel


kernel = _make_kernel()


# speedup vs baseline: 10.7526x; 1.1096x over previous
import dataclasses
import functools

import jax
import jax.numpy as jnp
from jax import lax
from jax.experimental import pallas as pl
from jax.experimental.pallas import tpu as pltpu
from jax.experimental.pallas import tpu_sc as plsc

N = 10000
E = 320000
D = 128

NC = 2
NS = 16
NW = NC * NS

CHUNK = 128
NCHUNK = E // CHUNK
FULL_G = NCHUNK // NW
EXTRA = NCHUNK - FULL_G * NW
ROWS_PER_TILE = 624

_ZV = 2000


NBUF = 3


def _sc_body(ei_hbm, ew_hbm, x_hbm, agg_out, wsum_out,
             agg_sh, wsum_sh,
             rows0, rows1, rows2, idx0, idx1, idx2, w0, w1, w2,
             sg0, sg1, sg2, sr0, sr1, sr2, sw0, sw1, sw2,
             si0, si1, si2):
    rows = (rows0, rows1, rows2)
    idx = (idx0, idx1, idx2)
    wv = (w0, w1, w2)
    sg = (sg0, sg1, sg2)
    sr = (sr0, sr1, sr2)
    sw = (sw0, sw1, sw2)
    si = (si0, si1, si2)
    rows_v = rows0

    c = lax.axis_index("c")
    s = lax.axis_index("s")
    wid = s * NC + c

    @pl.loop(0, CHUNK)
    def _(i):
        for m in range(D // 16):
            rows_v[i, pl.ds(m * 16, 16)] = jnp.zeros((16,), jnp.float32)

    for m in range(CHUNK // 16):
        w0[pl.ds(m * 16, 16)] = jnp.zeros((16,), jnp.float32)

    base_r = s * ROWS_PER_TILE
    off = 0
    for sz in (128, 128, 128, 128, 112):
        pltpu.sync_copy(rows_v.at[pl.ds(0, sz)],
                        agg_sh.at[pl.ds(base_r + off, sz)])
        off += sz

    @pl.when(s == NS - 1)
    def _():
        pltpu.sync_copy(rows_v.at[pl.ds(0, 16)],
                        agg_sh.at[pl.ds(NS * ROWS_PER_TILE, 16)])

    woff = 0
    for wsz in (128, 128, 128, 128, 112):
        pltpu.sync_copy(w0.at[pl.ds(0, wsz)],
                        wsum_sh.at[pl.ds(base_r + woff, wsz)])
        woff += wsz

    @pl.when(s == NS - 1)
    def _():
        pltpu.sync_copy(w0.at[pl.ds(0, 16)],
                        wsum_sh.at[pl.ds(NS * ROWS_PER_TILE, 16)])

    plsc.subcore_barrier()

    def idx_start(cid, b):
        base = cid * CHUNK
        pltpu.async_copy(ei_hbm.at[:, pl.ds(base, CHUNK)], idx[b], si[b])
        pltpu.async_copy(ew_hbm.at[pl.ds(base, CHUNK)], wv[b], si[b])

    def idx_wait(cid, b):
        base = cid * CHUNK
        pltpu.make_async_copy(ei_hbm.at[:, pl.ds(base, CHUNK)], idx[b],
                              si[b]).wait()
        pltpu.make_async_copy(ew_hbm.at[pl.ds(base, CHUNK)], wv[b],
                              si[b]).wait()

    def gather_start(b):
        pltpu.async_copy(x_hbm.at[idx[b].at[0]], rows[b], sg[b])

    def gather_wait(b):
        pltpu.make_async_copy(x_hbm.at[idx[b].at[0]], rows[b], sg[b]).wait()

    def scale(b):
        rb = rows[b]
        wb = wv[b]

        @plsc.parallel_loop(0, CHUNK, unroll=4)
        def _(i):
            wj = plsc.load_gather(wb, [jnp.full((16,), i, jnp.int32)])
            for m in range(D // 16):
                sl = pl.ds(m * 16, 16)
                rb[i, sl] = rb[i, sl] * wj

    def scatter_start(b):
        pltpu.async_copy(rows[b], agg_sh.at[idx[b].at[1]], sr[b], add=True)
        pltpu.async_copy(wv[b], wsum_sh.at[idx[b].at[1]], sw[b], add=True)

    def scatter_wait(b):
        pltpu.make_async_copy(rows[b], agg_sh.at[idx[b].at[1]], sr[b]).wait()
        pltpu.make_async_copy(wv[b], wsum_sh.at[idx[b].at[1]], sw[b]).wait()

    idx_start(wid, 0)
    idx_wait(wid, 0)
    gather_start(0)

    @pl.loop(0, FULL_G // NBUF)
    def _(G):
        for u in range(NBUF):
            cc = NBUF * G + u
            b = u
            b1 = (u + 1) % NBUF

            @pl.when(cc < FULL_G - 1)
            def _():
                @pl.when(cc >= 2)
                def _():
                    scatter_wait(b1)
                idx_start(wid + NW * (cc + 1), b1)

            gather_wait(b)

            @pl.when(cc < FULL_G - 1)
            def _():
                idx_wait(wid + NW * (cc + 1), b1)
                gather_start(b1)

            scale(b)
            scatter_start(b)

    @pl.when(wid < EXTRA)
    def _():
        scatter_wait(0)
        idx_start(FULL_G * NW + wid, 0)
        idx_wait(FULL_G * NW + wid, 0)
        gather_start(0)
        gather_wait(0)
        scale(0)
        scatter_start(0)

    scatter_wait(1)
    scatter_wait(2)
    scatter_wait(0)

    plsc.subcore_barrier()

    pltpu.sync_copy(agg_sh.at[pl.ds(base_r, ROWS_PER_TILE)],
                    agg_out.at[c, pl.ds(base_r, ROWS_PER_TILE)])

    @pl.when(s == NS - 1)
    def _():
        pltpu.sync_copy(agg_sh.at[pl.ds(NS * ROWS_PER_TILE, 16)],
                        agg_out.at[c, pl.ds(NS * ROWS_PER_TILE, 16)])

    @pl.when(s == 0)
    def _():
        pltpu.sync_copy(wsum_sh, wsum_out.at[c])


def _make_sc_call(interpret=False):
    mesh = plsc.VectorSubcoreMesh(core_axis_name="c", subcore_axis_name="s",
                                  num_cores=NC, num_subcores=NS)
    cp = pltpu.CompilerParams()
    if "needs_layout_passes" in pltpu.CompilerParams.__dataclass_fields__:
        cp = dataclasses.replace(cp, needs_layout_passes=False)
    return pl.kernel(
        _sc_body,
        out_type=(
            jax.ShapeDtypeStruct((NC, N, D), jnp.float32),
            jax.ShapeDtypeStruct((NC, N), jnp.float32),
        ),
        mesh=mesh,
        scratch_types=(
            pltpu.VMEM_SHARED((N, D), jnp.float32),
            pltpu.VMEM_SHARED((N,), jnp.float32),
            pltpu.VMEM((CHUNK, D), jnp.float32),
            pltpu.VMEM((CHUNK, D), jnp.float32),
            pltpu.VMEM((CHUNK, D), jnp.float32),
            pltpu.VMEM((2, CHUNK), jnp.int32),
            pltpu.VMEM((2, CHUNK), jnp.int32),
            pltpu.VMEM((2, CHUNK), jnp.int32),
            pltpu.VMEM((CHUNK,), jnp.float32),
            pltpu.VMEM((CHUNK,), jnp.float32),
            pltpu.VMEM((CHUNK,), jnp.float32),
            pltpu.SemaphoreType.DMA,
            pltpu.SemaphoreType.DMA,
            pltpu.SemaphoreType.DMA,
            pltpu.SemaphoreType.DMA,
            pltpu.SemaphoreType.DMA,
            pltpu.SemaphoreType.DMA,
            pltpu.SemaphoreType.DMA,
            pltpu.SemaphoreType.DMA,
            pltpu.SemaphoreType.DMA,
            pltpu.SemaphoreType.DMA,
            pltpu.SemaphoreType.DMA,
            pltpu.SemaphoreType.DMA,
        ),
        compiler_params=cp,
        interpret=interpret,
    )


R = 400
GRID = N // R


def _tc_body(pa_ref, pb_ref, sa_ref, sb_ref, w_ref, b_ref, o_ref):
    acc = pa_ref[0] + pb_ref[0]
    z = jnp.dot(acc, w_ref[...], preferred_element_type=jnp.float32,
                precision=lax.Precision.HIGHEST)
    svec = sa_ref[0, 0, 0] + sb_ref[0, 0, 0]
    z = z + svec[:, None] * b_ref[0][None, :]
    o_ref[...] = jnp.where(z > 0, z, jnp.exp(z) - 1.0)


def _make_tc_call(interpret=False):
    return pl.pallas_call(
        _tc_body,
        grid=(GRID,),
        in_specs=[
            pl.BlockSpec((1, R, D), lambda i: (0, i, 0)),
            pl.BlockSpec((1, R, D), lambda i: (1, i, 0)),
            pl.BlockSpec((1, 1, 1, R), lambda i: (0, i, 0, 0)),
            pl.BlockSpec((1, 1, 1, R), lambda i: (1, i, 0, 0)),
            pl.BlockSpec((D, D), lambda i: (0, 0)),
            pl.BlockSpec((1, D), lambda i: (0, 0)),
        ],
        out_specs=pl.BlockSpec((R, D), lambda i: (i, 0)),
        out_shape=jax.ShapeDtypeStruct((N, D), jnp.float32),
        interpret=interpret,
    )


def _make_kernel(interpret=False):
    sc_call = _make_sc_call(interpret)
    tc_call = _make_tc_call(interpret)

    @jax.jit
    def kernel(x, edge_index, edge_weight, W, b):
        agg_parts, wsum_parts = sc_call(edge_index, edge_weight, x)
        wsum_r = wsum_parts.reshape(NC, GRID, 1, R)
        return tc_call(agg_parts, agg_parts, wsum_r, wsum_r, W,
                       b.reshape(1, D))

    return kernel


kernel = _make_kernel()


# speedup vs baseline: 11.0209x; 1.0249x over previous
import dataclasses
import functools

import jax
import jax.numpy as jnp
from jax import lax
from jax.experimental import pallas as pl
from jax.experimental.pallas import tpu as pltpu
from jax.experimental.pallas import tpu_sc as plsc

N = 10000
E = 320000
D = 128

NC = 2
NS = 16
NW = NC * NS

CHUNK = 128
NCHUNK = E // CHUNK
FULL_G = NCHUNK // NW
EXTRA = NCHUNK - FULL_G * NW
ROWS_PER_TILE = 624

_ZV = 2000


NBUF = 3


def _sc_body(ei_hbm, ew_hbm, x_hbm, agg_out, wsum_out,
             agg_sh, wsum_sh,
             rows0, rows1, rows2, idx0, idx1, idx2, w0, w1, w2,
             sg0, sg1, sg2, sr0, sr1, sr2, sw0, sw1, sw2,
             si0, si1, si2):
    rows = (rows0, rows1, rows2)
    idx = (idx0, idx1, idx2)
    wv = (w0, w1, w2)
    sg = (sg0, sg1, sg2)
    sr = (sr0, sr1, sr2)
    sw = (sw0, sw1, sw2)
    si = (si0, si1, si2)
    rows_v = rows0

    c = lax.axis_index("c")
    s = lax.axis_index("s")
    wid = s * NC + c

    @pl.loop(0, CHUNK)
    def _(i):
        for m in range(D // 16):
            rows_v[i, pl.ds(m * 16, 16)] = jnp.zeros((16,), jnp.float32)

    for m in range(CHUNK // 16):
        w0[pl.ds(m * 16, 16)] = jnp.zeros((16,), jnp.float32)

    base_r = s * ROWS_PER_TILE
    off = 0
    for sz in (128, 128, 128, 128, 112):
        pltpu.sync_copy(rows_v.at[pl.ds(0, sz)],
                        agg_sh.at[pl.ds(base_r + off, sz)])
        off += sz

    @pl.when(s == NS - 1)
    def _():
        pltpu.sync_copy(rows_v.at[pl.ds(0, 16)],
                        agg_sh.at[pl.ds(NS * ROWS_PER_TILE, 16)])

    woff = 0
    for wsz in (128, 128, 128, 128, 112):
        pltpu.sync_copy(w0.at[pl.ds(0, wsz)],
                        wsum_sh.at[pl.ds(base_r + woff, wsz)])
        woff += wsz

    @pl.when(s == NS - 1)
    def _():
        pltpu.sync_copy(w0.at[pl.ds(0, 16)],
                        wsum_sh.at[pl.ds(NS * ROWS_PER_TILE, 16)])

    plsc.subcore_barrier()

    def idx_start(cid, b):
        base = cid * CHUNK
        pltpu.async_copy(ei_hbm.at[:, pl.ds(base, CHUNK)], idx[b], si[b])
        pltpu.async_copy(ew_hbm.at[pl.ds(base, CHUNK)], wv[b], si[b])

    def idx_wait(cid, b):
        base = cid * CHUNK
        pltpu.make_async_copy(ei_hbm.at[:, pl.ds(base, CHUNK)], idx[b],
                              si[b]).wait()
        pltpu.make_async_copy(ew_hbm.at[pl.ds(base, CHUNK)], wv[b],
                              si[b]).wait()

    def gather_start(b):
        pltpu.async_copy(x_hbm.at[idx[b].at[0]], rows[b], sg[b])

    def gather_wait(b):
        pltpu.make_async_copy(x_hbm.at[idx[b].at[0]], rows[b], sg[b]).wait()

    def scale(b):
        rb = rows[b]
        wb = wv[b]

        @plsc.parallel_loop(0, CHUNK, unroll=4)
        def _(i):
            wj = plsc.load_gather(wb, [jnp.full((16,), i, jnp.int32)])
            for m in range(D // 16):
                sl = pl.ds(m * 16, 16)
                rb[i, sl] = rb[i, sl] * wj

    def scatter_start(b):
        pltpu.async_copy(rows[b], agg_sh.at[idx[b].at[1]], sr[b], add=True)
        pltpu.async_copy(wv[b], wsum_sh.at[idx[b].at[1]], sw[b], add=True)

    def scatter_wait(b):
        pltpu.make_async_copy(rows[b], agg_sh.at[idx[b].at[1]], sr[b]).wait()
        pltpu.make_async_copy(wv[b], wsum_sh.at[idx[b].at[1]], sw[b]).wait()

    idx_start(wid, 0)
    idx_wait(wid, 0)
    gather_start(0)
    idx_start(wid + NW, 1)
    idx_wait(wid + NW, 1)
    gather_start(1)

    @pl.loop(0, FULL_G // NBUF)
    def _(G):
        for u in range(NBUF):
            cc = NBUF * G + u
            b = u
            b2 = (u + 2) % NBUF

            gather_wait(b)
            scale(b)
            scatter_start(b)

            @pl.when(cc < FULL_G - 2)
            def _():
                @pl.when(cc >= 1)
                def _():
                    scatter_wait(b2)
                idx_start(wid + NW * (cc + 2), b2)
                idx_wait(wid + NW * (cc + 2), b2)
                gather_start(b2)

    @pl.when(wid < EXTRA)
    def _():
        scatter_wait(0)
        idx_start(FULL_G * NW + wid, 0)
        idx_wait(FULL_G * NW + wid, 0)
        gather_start(0)
        gather_wait(0)
        scale(0)
        scatter_start(0)

    scatter_wait(1)
    scatter_wait(2)
    scatter_wait(0)

    plsc.subcore_barrier()

    pltpu.sync_copy(agg_sh.at[pl.ds(base_r, ROWS_PER_TILE)],
                    agg_out.at[c, pl.ds(base_r, ROWS_PER_TILE)])

    @pl.when(s == NS - 1)
    def _():
        pltpu.sync_copy(agg_sh.at[pl.ds(NS * ROWS_PER_TILE, 16)],
                        agg_out.at[c, pl.ds(NS * ROWS_PER_TILE, 16)])

    @pl.when(s == 0)
    def _():
        pltpu.sync_copy(wsum_sh, wsum_out.at[c])


def _make_sc_call(interpret=False):
    mesh = plsc.VectorSubcoreMesh(core_axis_name="c", subcore_axis_name="s",
                                  num_cores=NC, num_subcores=NS)
    cp = pltpu.CompilerParams()
    if "needs_layout_passes" in pltpu.CompilerParams.__dataclass_fields__:
        cp = dataclasses.replace(cp, needs_layout_passes=False)
    return pl.kernel(
        _sc_body,
        out_type=(
            jax.ShapeDtypeStruct((NC, N, D), jnp.float32),
            jax.ShapeDtypeStruct((NC, N), jnp.float32),
        ),
        mesh=mesh,
        scratch_types=(
            pltpu.VMEM_SHARED((N, D), jnp.float32),
            pltpu.VMEM_SHARED((N,), jnp.float32),
            pltpu.VMEM((CHUNK, D), jnp.float32),
            pltpu.VMEM((CHUNK, D), jnp.float32),
            pltpu.VMEM((CHUNK, D), jnp.float32),
            pltpu.VMEM((2, CHUNK), jnp.int32),
            pltpu.VMEM((2, CHUNK), jnp.int32),
            pltpu.VMEM((2, CHUNK), jnp.int32),
            pltpu.VMEM((CHUNK,), jnp.float32),
            pltpu.VMEM((CHUNK,), jnp.float32),
            pltpu.VMEM((CHUNK,), jnp.float32),
            pltpu.SemaphoreType.DMA,
            pltpu.SemaphoreType.DMA,
            pltpu.SemaphoreType.DMA,
            pltpu.SemaphoreType.DMA,
            pltpu.SemaphoreType.DMA,
            pltpu.SemaphoreType.DMA,
            pltpu.SemaphoreType.DMA,
            pltpu.SemaphoreType.DMA,
            pltpu.SemaphoreType.DMA,
            pltpu.SemaphoreType.DMA,
            pltpu.SemaphoreType.DMA,
            pltpu.SemaphoreType.DMA,
        ),
        compiler_params=cp,
        interpret=interpret,
    )


R = 400
GRID = N // R


def _tc_body(pa_ref, pb_ref, sa_ref, sb_ref, w_ref, b_ref, o_ref):
    acc = pa_ref[0] + pb_ref[0]
    z = jnp.dot(acc, w_ref[...], preferred_element_type=jnp.float32,
                precision=lax.Precision.HIGHEST)
    svec = sa_ref[0, 0, 0] + sb_ref[0, 0, 0]
    z = z + svec[:, None] * b_ref[0][None, :]
    o_ref[...] = jnp.where(z > 0, z, jnp.exp(z) - 1.0)


def _make_tc_call(interpret=False):
    return pl.pallas_call(
        _tc_body,
        grid=(GRID,),
        in_specs=[
            pl.BlockSpec((1, R, D), lambda i: (0, i, 0)),
            pl.BlockSpec((1, R, D), lambda i: (1, i, 0)),
            pl.BlockSpec((1, 1, 1, R), lambda i: (0, i, 0, 0)),
            pl.BlockSpec((1, 1, 1, R), lambda i: (1, i, 0, 0)),
            pl.BlockSpec((D, D), lambda i: (0, 0)),
            pl.BlockSpec((1, D), lambda i: (0, 0)),
        ],
        out_specs=pl.BlockSpec((R, D), lambda i: (i, 0)),
        out_shape=jax.ShapeDtypeStruct((N, D), jnp.float32),
        interpret=interpret,
    )


def _make_kernel(interpret=False):
    sc_call = _make_sc_call(interpret)
    tc_call = _make_tc_call(interpret)

    @jax.jit
    def kernel(x, edge_index, edge_weight, W, b):
        agg_parts, wsum_parts = sc_call(edge_index, edge_weight, x)
        wsum_r = wsum_parts.reshape(NC, GRID, 1, R)
        return tc_call(agg_parts, agg_parts, wsum_r, wsum_r, W,
                       b.reshape(1, D))

    return kernel


kernel = _make_kernel()
